# baseline (speedup 1.0000x reference)
import functools

import jax
import jax.numpy as jnp
from jax.experimental import pallas as pl
from jax.experimental.pallas import tpu as pltpu

_TOPK = 128
_ROWS_PER_BLOCK = 512


def _topk_mask_kernel(x_ref, o_ref, key_ref, *, k):
    rows, h, w = x_ref.shape
    hw = h * w
    x = x_ref[...].reshape(rows, hw)
    b = jax.lax.bitcast_convert_type(x, jnp.int32)
    key_ref[...] = b ^ jnp.where(b < 0, jnp.int32(0x7FFFFFFF), jnp.int32(0))
    lo0 = jnp.full((rows, 1), jnp.iinfo(jnp.int32).min, jnp.int32)
    hi0 = jnp.full((rows, 1), jnp.iinfo(jnp.int32).max, jnp.int32)

    def body(_, carry):
        lo, hi = carry
        mid = (lo >> 1) + (hi >> 1) + (lo & hi & jnp.int32(1))
        cnt = jnp.sum((key_ref[...] >= mid).astype(jnp.int32), axis=1, keepdims=True)
        ge = cnt >= k
        return jnp.where(ge, mid, lo), jnp.where(ge, hi, mid)

    lo, _ = jax.lax.fori_loop(0, 32, body, (lo0, hi0))
    out = jnp.where(key_ref[...] >= lo, x, jnp.float32(0))
    o_ref[...] = out.reshape(rows, h, w)


def kernel(x):
    n, c, h, w = x.shape
    rows = n * c
    hw = h * w
    xr = x.reshape(rows, h, w)
    out = pl.pallas_call(
        functools.partial(_topk_mask_kernel, k=_TOPK),
        grid=(rows // _ROWS_PER_BLOCK,),
        in_specs=[pl.BlockSpec((_ROWS_PER_BLOCK, h, w), lambda i: (i, 0, 0))],
        out_specs=pl.BlockSpec((_ROWS_PER_BLOCK, h, w), lambda i: (i, 0, 0)),
        out_shape=jax.ShapeDtypeStruct((rows, h, w), x.dtype),
        scratch_shapes=[pltpu.VMEM((_ROWS_PER_BLOCK, hw), jnp.int32)],
    )(xr)
    return out.reshape(n, c, h, w)


# speedup vs baseline: 2.2463x; 2.2463x over previous
import functools

import jax
import jax.numpy as jnp
from jax.experimental import pallas as pl
from jax.experimental.pallas import tpu as pltpu

_TOPK = 128
_ROWS_PER_BLOCK = 512


def _topk_mask_kernel(x_ref, o_ref, keyt_ref, *, k):
    x = x_ref[...]
    rows = x.shape[0]
    xt = x.T
    bt = jax.lax.bitcast_convert_type(xt, jnp.int32)
    keyt_ref[...] = bt ^ jnp.where(bt < 0, jnp.int32(0x7FFFFFFF), jnp.int32(0))
    lo0 = jnp.full((1, rows), jnp.iinfo(jnp.int32).min, jnp.int32)
    hi0 = jnp.full((1, rows), jnp.iinfo(jnp.int32).max, jnp.int32)

    def body(_, carry):
        lo, hi = carry
        mid = (lo >> 1) + (hi >> 1) + (lo & hi & jnp.int32(1))
        cnt = jnp.sum(
            (keyt_ref[...] >= mid).astype(jnp.int32), axis=0, keepdims=True
        )
        ge = cnt >= k
        return jnp.where(ge, mid, lo), jnp.where(ge, hi, mid)

    lo, _ = jax.lax.fori_loop(0, 32, body, (lo0, hi0))
    lo_col = lo.T
    b = jax.lax.bitcast_convert_type(x, jnp.int32)
    key = b ^ jnp.where(b < 0, jnp.int32(0x7FFFFFFF), jnp.int32(0))
    o_ref[...] = jnp.where(key >= lo_col, x, jnp.float32(0))


def kernel(x):
    n, c, h, w = x.shape
    rows = n * c
    hw = h * w
    xr = x.reshape(rows, hw)
    out = pl.pallas_call(
        functools.partial(_topk_mask_kernel, k=_TOPK),
        grid=(rows // _ROWS_PER_BLOCK,),
        in_specs=[pl.BlockSpec((_ROWS_PER_BLOCK, hw), lambda i: (i, 0))],
        out_specs=pl.BlockSpec((_ROWS_PER_BLOCK, hw), lambda i: (i, 0)),
        out_shape=jax.ShapeDtypeStruct((rows, hw), x.dtype),
        scratch_shapes=[pltpu.VMEM((hw, _ROWS_PER_BLOCK), jnp.int32)],
    )(xr)
    return out.reshape(n, c, h, w)


# speedup vs baseline: 2.2866x; 1.0179x over previous
import functools

import jax
import jax.numpy as jnp
from jax.experimental import pallas as pl
from jax.experimental.pallas import tpu as pltpu

_TOPK = 128
_ROWS_PER_BLOCK = 1024


def _topk_mask_kernel(x_ref, o_ref, keyt_ref, *, k):
    x = x_ref[...]
    rows = x.shape[0]
    xt = x.T
    bt = jax.lax.bitcast_convert_type(xt, jnp.int32)
    keyt_ref[...] = bt ^ jnp.where(bt < 0, jnp.int32(0x7FFFFFFF), jnp.int32(0))
    lo0 = jnp.full((1, rows), jnp.iinfo(jnp.int32).min, jnp.int32)
    hi0 = jnp.full((1, rows), jnp.iinfo(jnp.int32).max, jnp.int32)

    def body(_, carry):
        lo, hi = carry
        mid = (lo >> 1) + (hi >> 1) + (lo & hi & jnp.int32(1))
        cnt = jnp.sum(
            (keyt_ref[...] >= mid).astype(jnp.int32), axis=0, keepdims=True
        )
        ge = cnt >= k
        return jnp.where(ge, mid, lo), jnp.where(ge, hi, mid)

    lo, _ = jax.lax.fori_loop(0, 32, body, (lo0, hi0))
    lo_col = lo.T
    b = jax.lax.bitcast_convert_type(x, jnp.int32)
    key = b ^ jnp.where(b < 0, jnp.int32(0x7FFFFFFF), jnp.int32(0))
    o_ref[...] = jnp.where(key >= lo_col, x, jnp.float32(0))


def kernel(x):
    n, c, h, w = x.shape
    rows = n * c
    hw = h * w
    xr = x.reshape(rows, hw)
    out = pl.pallas_call(
        functools.partial(_topk_mask_kernel, k=_TOPK),
        grid=(rows // _ROWS_PER_BLOCK,),
        in_specs=[pl.BlockSpec((_ROWS_PER_BLOCK, hw), lambda i: (i, 0))],
        out_specs=pl.BlockSpec((_ROWS_PER_BLOCK, hw), lambda i: (i, 0)),
        out_shape=jax.ShapeDtypeStruct((rows, hw), x.dtype),
        scratch_shapes=[pltpu.VMEM((hw, _ROWS_PER_BLOCK), jnp.int32)],
    )(xr)
    return out.reshape(n, c, h, w)
